# baseline (speedup 1.0000x reference)
import functools

import jax
import jax.numpy as jnp
from jax import lax
from jax.experimental import pallas as pl
from jax.experimental.pallas import tpu as pltpu
from jax.experimental.pallas import tpu_sc as plsc

N = 10000
D = 128
E = 320000
NC = 2
NS = 16
L = 16
C = 64
CHUNKS = -(-E // C)
CPW = -(-CHUNKS // (NC * NS * 8)) * 8
CHUNKS_PAD = CPW * NC * NS
E_PAD = CHUNKS_PAD * C
ROWS_PER_TILE = -(-N // (NS * 8)) * 8
AGG_ROWS = ROWS_PER_TILE * NS
IB = 16
NB = CPW // IB
NBUF = 4


def _make_sc_agg(scale: bool):
    mesh = plsc.VectorSubcoreMesh(core_axis_name="c", subcore_axis_name="s",
                                  num_cores=NC, num_subcores=NS)

    @functools.partial(
        pl.kernel,
        mesh=mesh,
        out_type=jax.ShapeDtypeStruct((NC, AGG_ROWS, D), jnp.float32),
        scratch_types=[
            pltpu.VMEM((2, IB, C), jnp.int32),
            pltpu.VMEM((2, IB, C), jnp.int32),
            pltpu.VMEM((2, IB, C), jnp.float32),
            pltpu.VMEM((NBUF, C, D), jnp.float32),
            pltpu.VMEM_SHARED((AGG_ROWS, D), jnp.float32),
        ] + [pltpu.SemaphoreType.DMA] * (2 * NBUF + 1),
    )
    def k(x_hbm, src_hbm, dst_hbm, ew_hbm, zeros_hbm, out_hbm,
          src_i, dst_i, ew_i, rows_v, agg_sh, *sems):
        c = lax.axis_index("c")
        s = lax.axis_index("s")
        base_row = s * ROWS_PER_TILE
        gsem = sems[:NBUF]
        ssem = sems[NBUF:2 * NBUF]
        isem = sems[2 * NBUF]

        pltpu.sync_copy(zeros_hbm.at[pl.ds(0, ROWS_PER_TILE)],
                        agg_sh.at[pl.ds(base_row, ROWS_PER_TILE)])

        plsc.subcore_barrier()

        cb = (c * NS + s) * CPW

        def fire_idx(bb, st):
            off = pl.ds(cb + bb * IB, IB)
            pltpu.async_copy(src_hbm.at[off], src_i.at[st], isem)
            pltpu.async_copy(dst_hbm.at[off], dst_i.at[st], isem)
            if scale:
                pltpu.async_copy(ew_hbm.at[off], ew_i.at[st], isem)

        def drain_idx(st, bb):
            off = pl.ds(cb + bb * IB, IB)
            pltpu.make_async_copy(src_hbm.at[off], src_i.at[st], isem).wait()
            pltpu.make_async_copy(dst_hbm.at[off], dst_i.at[st], isem).wait()
            if scale:
                pltpu.make_async_copy(ew_hbm.at[off], ew_i.at[st],
                                      isem).wait()

        def fire_gather(g, st, r):
            pltpu.async_copy(x_hbm.at[src_i.at[st, r]], rows_v.at[g],
                             gsem[g])

        def drain_rows(sem, g):
            pltpu.make_async_copy(x_hbm.at[pl.ds(0, C)], rows_v.at[g],
                                  sem).wait()

        def scale_buf(g, st, r):
            for gi in range(C // L):
                ew16 = ew_i[st, r, pl.ds(gi * L, L)]
                for lane in range(L):
                    i = gi * L + lane
                    ewb = jnp.full((L,), ew16[lane], jnp.float32)
                    for f in range(D // L):
                        sl = pl.ds(f * L, L)
                        rows_v[g, i, sl] = rows_v[g, i, sl] * ewb

        def fire_scatter(g, st, r):
            pltpu.async_copy(rows_v.at[g], agg_sh.at[dst_i.at[st, r]],
                             ssem[g], add=True)

        def process_batch(st, bb):
            drain_idx(st, bb)

            @pl.when(bb + 1 < NB)
            def _():
                fire_idx(bb + 1, 1 - st)

            for g in range(NBUF):
                fire_gather(g, st, g)

            def inner(t, carry):
                for g in range(NBUF):
                    r = NBUF * t + g
                    drain_rows(gsem[g], g)
                    if scale:
                        scale_buf(g, st, r)
                    fire_scatter(g, st, r)
                for g in range(NBUF):
                    drain_rows(ssem[g], g)

                    @pl.when(NBUF * t + NBUF + g < IB)
                    def _():
                        fire_gather(g, st, NBUF * t + NBUF + g)
                return carry

            lax.fori_loop(0, IB // NBUF, inner, 0)

        fire_idx(0, 0)

        def outer(b2, carry):
            process_batch(0, 2 * b2)
            process_batch(1, 2 * b2 + 1)
            return carry

        lax.fori_loop(0, NB // 2, outer, 0)
        plsc.subcore_barrier()

        pltpu.sync_copy(agg_sh.at[pl.ds(base_row, ROWS_PER_TILE)],
                        out_hbm.at[c, pl.ds(base_row, ROWS_PER_TILE)])

    return k


_sc_agg_scaled = _make_sc_agg(True)
_sc_agg_plain = _make_sc_agg(False)

BR = 2000


def _dense_body(a_ref, x_ref, wr_ref, wt_ref, b_ref, o_ref, *, relu):
    dn = (((1,), (0,)), ((), ()))
    a = a_ref[0] + a_ref[1]
    acc = lax.dot_general(a, wr_ref[...], dn,
                          preferred_element_type=jnp.float32,
                          precision=lax.Precision.HIGHEST)
    acc = acc + lax.dot_general(x_ref[...], wt_ref[...], dn,
                                preferred_element_type=jnp.float32,
                                precision=lax.Precision.HIGHEST)
    acc = acc + b_ref[...]
    if relu:
        acc = jnp.maximum(acc, 0.0)
    o_ref[...] = acc


def _dense(agg2, x, wr, wt, b, relu):
    return pl.pallas_call(
        functools.partial(_dense_body, relu=relu),
        grid=(N // BR,),
        in_specs=[
            pl.BlockSpec((NC, BR, D), lambda i: (0, i, 0)),
            pl.BlockSpec((BR, D), lambda i: (i, 0)),
            pl.BlockSpec((D, D), lambda i: (0, 0)),
            pl.BlockSpec((D, D), lambda i: (0, 0)),
            pl.BlockSpec((1, D), lambda i: (0, 0)),
        ],
        out_specs=pl.BlockSpec((BR, D), lambda i: (i, 0)),
        out_shape=jax.ShapeDtypeStruct((N, D), jnp.float32),
    )(agg2, x, wr, wt, b.reshape(1, D))


def kernel(x, edge_index, edge_weight,
           W1_rel, b1_rel, W1_root,
           W2_rel, b2_rel, W2_root,
           W3_rel, b3_rel, W3_root):
    pad = E_PAD - E
    pad_src = jnp.arange(pad, dtype=jnp.int32) % N
    pad_dst = N + jnp.arange(pad, dtype=jnp.int32) % (AGG_ROWS - N)
    src = jnp.concatenate([edge_index[0], pad_src]).reshape(CHUNKS_PAD, C)
    dst = jnp.concatenate([edge_index[1], pad_dst]).reshape(CHUNKS_PAD, C)
    ew = jnp.concatenate(
        [edge_weight, jnp.zeros((pad,), jnp.float32)]).reshape(CHUNKS_PAD, C)
    zeros = jnp.zeros((ROWS_PER_TILE, D), jnp.float32)
    del pad

    agg = _sc_agg_scaled(x, src, dst, ew, zeros)
    h = _dense(agg, x, W1_rel, W1_root, b1_rel, relu=True)
    agg = _sc_agg_scaled(h, src, dst, ew, zeros)
    h = _dense(agg, h, W2_rel, W2_root, b2_rel, relu=True)
    agg = _sc_agg_plain(h, src, dst, ew, zeros)
    out = _dense(agg, h, W3_rel, W3_root, b3_rel, relu=False)
    return out


# speedup vs baseline: 1.5056x; 1.5056x over previous
import functools

import jax
import jax.numpy as jnp
from jax import lax
from jax.experimental import pallas as pl
from jax.experimental.pallas import tpu as pltpu
from jax.experimental.pallas import tpu_sc as plsc

N = 10000
D = 128
E = 320000
NC = 2
NS = 16
L = 16
C = 64
CHUNKS = -(-E // C)
CPW = -(-CHUNKS // (NC * NS * 8)) * 8
CHUNKS_PAD = CPW * NC * NS
E_PAD = CHUNKS_PAD * C
ROWS_PER_TILE = -(-N // (NS * 8)) * 8
AGG_ROWS = ROWS_PER_TILE * NS
IB = 16
NB = CPW // IB
NBUF = 4


def _make_sc_agg(scale: bool):
    mesh = plsc.VectorSubcoreMesh(core_axis_name="c", subcore_axis_name="s",
                                  num_cores=NC, num_subcores=NS)

    @functools.partial(
        pl.kernel,
        mesh=mesh,
        out_type=jax.ShapeDtypeStruct((NC, AGG_ROWS, D), jnp.float32),
        scratch_types=[
            pltpu.VMEM((2, IB, C), jnp.int32),
            pltpu.VMEM((2, IB, C), jnp.int32),
            pltpu.VMEM((2, IB, C), jnp.float32),
            pltpu.VMEM((NBUF, C, D), jnp.float32),
            pltpu.VMEM_SHARED((AGG_ROWS, D), jnp.float32),
        ] + [pltpu.SemaphoreType.DMA] * (2 * NBUF + 1),
    )
    def k(x_hbm, src_hbm, dst_hbm, ew_hbm, zeros_hbm, out_hbm,
          src_i, dst_i, ew_i, rows_v, agg_sh, *sems):
        c = lax.axis_index("c")
        s = lax.axis_index("s")
        base_row = s * ROWS_PER_TILE
        gsem = sems[:NBUF]
        ssem = sems[NBUF:2 * NBUF]
        isem = sems[2 * NBUF]

        pltpu.sync_copy(zeros_hbm.at[pl.ds(0, ROWS_PER_TILE)],
                        agg_sh.at[pl.ds(base_row, ROWS_PER_TILE)])

        plsc.subcore_barrier()

        cb = (c * NS + s) * CPW

        def fire_idx(bb, st):
            off = pl.ds(cb + bb * IB, IB)
            pltpu.async_copy(src_hbm.at[off], src_i.at[st], isem)
            pltpu.async_copy(dst_hbm.at[off], dst_i.at[st], isem)
            if scale:
                pltpu.async_copy(ew_hbm.at[off], ew_i.at[st], isem)

        def drain_idx(st, bb):
            off = pl.ds(cb + bb * IB, IB)
            pltpu.make_async_copy(src_hbm.at[off], src_i.at[st], isem).wait()
            pltpu.make_async_copy(dst_hbm.at[off], dst_i.at[st], isem).wait()
            if scale:
                pltpu.make_async_copy(ew_hbm.at[off], ew_i.at[st],
                                      isem).wait()

        def fire_gather(g, st, r):
            pltpu.async_copy(x_hbm.at[src_i.at[st, r]], rows_v.at[g],
                             gsem[g])

        def drain_rows(sem, g):
            pltpu.make_async_copy(x_hbm.at[pl.ds(0, C)], rows_v.at[g],
                                  sem).wait()

        def scale_buf(g, st, r):
            def grp_body(gi, carry2):
                ew16 = ew_i[st, r, pl.ds(gi * L, L)]
                for lane in range(L):
                    i = gi * L + lane
                    ewb = jnp.full((L,), ew16[lane], jnp.float32)
                    for f in range(D // L):
                        sl = pl.ds(f * L, L)
                        rows_v[g, i, sl] = rows_v[g, i, sl] * ewb
                return carry2
            lax.fori_loop(0, C // L, grp_body, 0)

        def fire_scatter(g, st, r):
            pltpu.async_copy(rows_v.at[g], agg_sh.at[dst_i.at[st, r]],
                             ssem[g], add=True)

        def process_batch(st, bb):
            drain_idx(st, bb)

            @pl.when(bb + 1 < NB)
            def _():
                fire_idx(bb + 1, 1 - st)

            for g in range(NBUF):
                fire_gather(g, st, g)

            def inner(t, carry):
                for g in range(NBUF):
                    r = NBUF * t + g
                    drain_rows(gsem[g], g)
                    if scale:
                        scale_buf(g, st, r)
                    fire_scatter(g, st, r)
                for g in range(NBUF):
                    drain_rows(ssem[g], g)

                    @pl.when(NBUF * t + NBUF + g < IB)
                    def _():
                        fire_gather(g, st, NBUF * t + NBUF + g)
                return carry

            lax.fori_loop(0, IB // NBUF, inner, 0)

        fire_idx(0, 0)

        def outer(b2, carry):
            process_batch(0, 2 * b2)
            process_batch(1, 2 * b2 + 1)
            return carry

        lax.fori_loop(0, NB // 2, outer, 0)
        plsc.subcore_barrier()

        pltpu.sync_copy(agg_sh.at[pl.ds(base_row, ROWS_PER_TILE)],
                        out_hbm.at[c, pl.ds(base_row, ROWS_PER_TILE)])

    return k


_sc_agg_scaled = _make_sc_agg(True)
_sc_agg_plain = _make_sc_agg(False)

BR = 2000


def _dense_body(a_ref, x_ref, wr_ref, wt_ref, b_ref, o_ref, *, relu):
    dn = (((1,), (0,)), ((), ()))
    a = a_ref[0] + a_ref[1]
    acc = lax.dot_general(a, wr_ref[...], dn,
                          preferred_element_type=jnp.float32,
                          precision=lax.Precision.HIGHEST)
    acc = acc + lax.dot_general(x_ref[...], wt_ref[...], dn,
                                preferred_element_type=jnp.float32,
                                precision=lax.Precision.HIGHEST)
    acc = acc + b_ref[...]
    if relu:
        acc = jnp.maximum(acc, 0.0)
    o_ref[...] = acc


def _dense(agg2, x, wr, wt, b, relu):
    return pl.pallas_call(
        functools.partial(_dense_body, relu=relu),
        grid=(N // BR,),
        in_specs=[
            pl.BlockSpec((NC, BR, D), lambda i: (0, i, 0)),
            pl.BlockSpec((BR, D), lambda i: (i, 0)),
            pl.BlockSpec((D, D), lambda i: (0, 0)),
            pl.BlockSpec((D, D), lambda i: (0, 0)),
            pl.BlockSpec((1, D), lambda i: (0, 0)),
        ],
        out_specs=pl.BlockSpec((BR, D), lambda i: (i, 0)),
        out_shape=jax.ShapeDtypeStruct((N, D), jnp.float32),
    )(agg2, x, wr, wt, b.reshape(1, D))


def kernel(x, edge_index, edge_weight,
           W1_rel, b1_rel, W1_root,
           W2_rel, b2_rel, W2_root,
           W3_rel, b3_rel, W3_root):
    pad = E_PAD - E
    pad_src = jnp.arange(pad, dtype=jnp.int32) % N
    pad_dst = N + jnp.arange(pad, dtype=jnp.int32) % (AGG_ROWS - N)
    src = jnp.concatenate([edge_index[0], pad_src]).reshape(CHUNKS_PAD, C)
    dst = jnp.concatenate([edge_index[1], pad_dst]).reshape(CHUNKS_PAD, C)
    ew = jnp.concatenate(
        [edge_weight, jnp.zeros((pad,), jnp.float32)]).reshape(CHUNKS_PAD, C)
    zeros = jnp.zeros((ROWS_PER_TILE, D), jnp.float32)
    del pad

    agg = _sc_agg_scaled(x, src, dst, ew, zeros)
    h = _dense(agg, x, W1_rel, W1_root, b1_rel, relu=True)
    agg = _sc_agg_scaled(h, src, dst, ew, zeros)
    h = _dense(agg, h, W2_rel, W2_root, b2_rel, relu=True)
    agg = _sc_agg_plain(h, src, dst, ew, zeros)
    out = _dense(agg, h, W3_rel, W3_root, b3_rel, relu=False)
    return out
